# baseline (speedup 1.0000x reference)
import functools

import jax
import jax.numpy as jnp
from jax import lax
from jax.experimental import pallas as pl
from jax.experimental.pallas import tpu as pltpu
from jax.experimental.pallas import tpu_sc as plsc

_DIM = 512
_L = 77
_XPOS = 5
_LPAD = 96
_NC = 2
_NS = 16
_NW = _NC * _NS
_REP = 64
_PLANES_PER_TILE = 3


def _sc_compose(idx_pad, s_star, table, bsz):
    parts = bsz // _REP
    rows_per_tile = bsz // _NW

    @functools.partial(
        pl.kernel,
        out_type=(
            jax.ShapeDtypeStruct((_L, bsz, _DIM), jnp.float32),
            jax.ShapeDtypeStruct((_L, bsz), jnp.int32),
        ),
        mesh=plsc.VectorSubcoreMesh(
            core_axis_name="c", subcore_axis_name="s",
            num_cores=_NC, num_subcores=_NS),
        scratch_types=[
            pltpu.VMEM((_LPAD,), jnp.int32),
            pltpu.VMEM((_LPAD, _DIM), jnp.float32),
            [pltpu.VMEM((_REP, _DIM), jnp.float32) for _ in range(2)],
            [pltpu.VMEM((bsz,), jnp.int32) for _ in range(_PLANES_PER_TILE)],
            pltpu.SemaphoreType.DMA,
            pltpu.SemaphoreType.DMA,
            pltpu.SemaphoreType.DMA,
            pltpu.SemaphoreType.DMA,
        ],
    )
    def k(idx_hbm, sstar_hbm, table_hbm, out_hbm, tokb_hbm,
          idx_v, stage, reps, tokreps, sem_g, sem_r, sem_m, sem_t):
        wid = lax.axis_index("s") * _NC + lax.axis_index("c")
        base = wid * rows_per_tile

        pltpu.sync_copy(idx_hbm, idx_v)
        c_emb = pltpu.async_copy(table_hbm.at[idx_v], stage, sem_g)
        relay = []
        for h in range(2):
            pltpu.async_copy(
                sstar_hbm.at[pl.ds(base + _REP * h, _REP)],
                reps[h], sem_t).wait()
            relay.append(pltpu.async_copy(
                reps[h], out_hbm.at[_XPOS, pl.ds(base + _REP * h, _REP)],
                sem_r))
        c_emb.wait()

        def fill_rep(p, rep):
            vecs = [stage[p, pl.ds(16 * i, 16)] for i in range(_DIM // 16)]

            def body(r, carry):
                for i, v in enumerate(vecs):
                    rep[r, pl.ds(16 * i, 16)] = v
                return carry

            lax.fori_loop(0, _REP, body, 0)

        tok_state = []
        plane_state = {}
        for kk in range(_PLANES_PER_TILE):
            p = wid + _NW * kk
            pred = jnp.logical_and(p < _L, p != _XPOS)
            rep = reps[kk % 2]

            if kk < 2:
                relay[kk].wait()
            else:
                prev_pred, prev_pend = plane_state[kk - 2]

                @pl.when(prev_pred)
                def _():
                    for d in prev_pend:
                        d.wait()

            pend = []

            @pl.when(pred)
            def _():
                fill_rep(p, rep)
                for j in range(parts):
                    pend.append(pltpu.async_copy(
                        rep, out_hbm.at[p, pl.ds(_REP * j, _REP)], sem_m))

            plane_state[kk] = (pred, pend)

            tpred = p < _L
            tpend = []

            @pl.when(tpred)
            def _():
                tvec = idx_v[pl.ds(p, 16)]
                tsplat = jnp.full((16,), tvec[0], dtype=jnp.int32)
                tokrep = tokreps[kk]

                def tbody(r, carry):
                    tokrep[pl.ds(16 * r, 16)] = tsplat
                    return carry

                lax.fori_loop(0, bsz // 16, tbody, 0)
                tpend.append(pltpu.async_copy(tokrep, tokb_hbm.at[p], sem_t))

            tok_state.append((tpred, tpend))

        for kk in range(_PLANES_PER_TILE - 2, _PLANES_PER_TILE):
            pred, pend = plane_state[kk]

            @pl.when(pred)
            def _():
                for d in pend:
                    d.wait()

        for tpred, tpend in tok_state:
            @pl.when(tpred)
            def _():
                for d in tpend:
                    d.wait()

    return k(idx_pad, s_star, table)


def kernel(s_star, table, tokenized):
    bsz = s_star.shape[0]
    tok = tokenized.reshape(_L).astype(jnp.int32)
    idx_pad = jnp.pad(tok, (0, _LPAD - _L))
    p77, t77 = _sc_compose(idx_pad, s_star.astype(jnp.float32), table, bsz)
    return jnp.transpose(p77, (1, 0, 2)), jnp.transpose(t77, (1, 0))


# speedup vs baseline: 1.0106x; 1.0106x over previous
import functools

import jax
import jax.numpy as jnp
from jax import lax
from jax.experimental import pallas as pl
from jax.experimental.pallas import tpu as pltpu
from jax.experimental.pallas import tpu_sc as plsc

_DIM = 512
_L = 77
_XPOS = 5
_LPAD = 96
_NC = 2
_NS = 16
_NW = _NC * _NS
_REP = 128
_PLANES_PER_TILE = 3


def _sc_compose(idx_pad, s_star, table, bsz):
    parts = bsz // _REP
    rows_per_tile = bsz // _NW

    @functools.partial(
        pl.kernel,
        out_type=(
            jax.ShapeDtypeStruct((_L, bsz, _DIM), jnp.float32),
            jax.ShapeDtypeStruct((_L, bsz), jnp.int32),
        ),
        mesh=plsc.VectorSubcoreMesh(
            core_axis_name="c", subcore_axis_name="s",
            num_cores=_NC, num_subcores=_NS),
        scratch_types=[
            pltpu.VMEM((_LPAD,), jnp.int32),
            pltpu.VMEM((_LPAD, _DIM), jnp.float32),
            pltpu.VMEM((_REP, _DIM), jnp.float32),
            [pltpu.VMEM((bsz,), jnp.int32) for _ in range(_PLANES_PER_TILE)],
            pltpu.SemaphoreType.DMA,
            pltpu.SemaphoreType.DMA,
            pltpu.SemaphoreType.DMA,
            pltpu.SemaphoreType.DMA,
        ],
    )
    def k(idx_hbm, sstar_hbm, table_hbm, out_hbm, tokb_hbm,
          idx_v, stage, rep, tokreps, sem_g, sem_r, sem_m, sem_t):
        wid = lax.axis_index("s") * _NC + lax.axis_index("c")
        base = wid * rows_per_tile

        pltpu.sync_copy(idx_hbm, idx_v)
        c_emb = pltpu.async_copy(table_hbm.at[idx_v], stage, sem_g)
        pltpu.async_copy(
            sstar_hbm.at[pl.ds(base, _REP)], rep, sem_t).wait()
        relay = pltpu.async_copy(
            rep, out_hbm.at[_XPOS, pl.ds(base, _REP)], sem_r)
        c_emb.wait()

        def fill_rep(p, rep):
            vecs = [stage[p, pl.ds(16 * i, 16)] for i in range(_DIM // 16)]

            def body(r, carry):
                for i, v in enumerate(vecs):
                    rep[r, pl.ds(16 * i, 16)] = v
                return carry

            lax.fori_loop(0, _REP, body, 0)

        tok_state = []
        plane_state = {}
        for kk in range(_PLANES_PER_TILE):
            p = wid + _NW * kk
            pred = jnp.logical_and(p < _L, p != _XPOS)

            if kk == 0:
                relay.wait()
            else:
                prev_pred, prev_pend = plane_state[kk - 1]

                @pl.when(prev_pred)
                def _():
                    for d in prev_pend:
                        d.wait()

            pend = []

            @pl.when(pred)
            def _():
                fill_rep(p, rep)
                for j in range(parts):
                    pend.append(pltpu.async_copy(
                        rep, out_hbm.at[p, pl.ds(_REP * j, _REP)], sem_m))

            plane_state[kk] = (pred, pend)

            tpred = p < _L
            tpend = []

            @pl.when(tpred)
            def _():
                tvec = idx_v[pl.ds(p, 16)]
                tsplat = jnp.full((16,), tvec[0], dtype=jnp.int32)
                tokrep = tokreps[kk]

                def tbody(r, carry):
                    tokrep[pl.ds(16 * r, 16)] = tsplat
                    return carry

                lax.fori_loop(0, bsz // 16, tbody, 0)
                tpend.append(pltpu.async_copy(tokrep, tokb_hbm.at[p], sem_t))

            tok_state.append((tpred, tpend))

        for kk in range(_PLANES_PER_TILE - 1, _PLANES_PER_TILE):
            pred, pend = plane_state[kk]

            @pl.when(pred)
            def _():
                for d in pend:
                    d.wait()

        for tpred, tpend in tok_state:
            @pl.when(tpred)
            def _():
                for d in tpend:
                    d.wait()

    return k(idx_pad, s_star, table)


def kernel(s_star, table, tokenized):
    bsz = s_star.shape[0]
    tok = tokenized.reshape(_L).astype(jnp.int32)
    idx_pad = jnp.pad(tok, (0, _LPAD - _L))
    p77, t77 = _sc_compose(idx_pad, s_star.astype(jnp.float32), table, bsz)
    return jnp.transpose(p77, (1, 0, 2)), jnp.transpose(t77, (1, 0))
